# baseline (speedup 1.0000x reference)
import functools

import jax
import jax.numpy as jnp
from jax import lax
from jax.experimental import pallas as pl
from jax.experimental.pallas import tpu as pltpu
from jax.experimental.pallas import tpu_sc as plsc

N = 10000
E = 320000
D_IN = 128
H = 128
H2 = 64
G = 128
OUT = 10

NP = 10240
PAD = NP - N
NC = 2
NS = 16
LANES = 16
SUB = 128
NSUB = 2
CH = NSUB * SUB
EPT = 20480
E_PAD = EPT * NS
NCHUNK = EPT // CH
DW = 80


def _lrelu(v):
    return jnp.where(v > 0, v, 0.2 * v)


def _fold(hhalf):
    n = hhalf.shape[0]
    blk = jnp.concatenate(
        [hhalf, jnp.ones((n, 1), jnp.float32),
         jnp.zeros((n, DW - H2 - 1), jnp.float32)], axis=1)
    return jnp.concatenate([blk, jnp.zeros((PAD, DW), jnp.float32)], axis=0)


def _fold_init(num0half, wself):
    n = num0half.shape[0]
    blk = jnp.concatenate(
        [num0half, wself, jnp.zeros((n, DW - H2 - 1), jnp.float32)], axis=1)
    return jnp.concatenate([blk, jnp.zeros((PAD, DW), jnp.float32)], axis=0)


def _pad_col(v):
    return jnp.concatenate([v, jnp.zeros((PAD, 1), jnp.float32)], axis=0)


def _prep1_body(x_ref, w_ref, as_ref, ad_ref,
                h_out, inum_out, as_out, ad_out, maxs_out):
    h = jnp.dot(x_ref[...], w_ref[...], preferred_element_type=jnp.float32)
    a_s = jnp.dot(h, as_ref[...].reshape(H, 1))
    a_d = jnp.dot(h, ad_ref[...].reshape(H, 1))
    maxs = jnp.max(a_s)
    c = _lrelu(maxs + a_d)
    wself = jnp.exp(_lrelu(a_s + a_d) - c)
    num0 = wself * h
    h_out[0] = _fold(h[:, :H // 2])
    h_out[1] = _fold(h[:, H // 2:])
    inum_out[0] = _fold_init(num0[:, :H // 2], wself)
    inum_out[1] = _fold_init(num0[:, H // 2:], wself)
    asp = _pad_col(a_s)
    as_out[0] = asp
    as_out[1] = asp
    ad_out[...] = _pad_col(a_d)
    maxs_out[...] = jnp.full((1, LANES), maxs, jnp.float32)


_prep1 = pl.pallas_call(
    _prep1_body,
    compiler_params=pltpu.CompilerParams(vmem_limit_bytes=100 * 1024 * 1024),
    out_shape=(
        jax.ShapeDtypeStruct((2, NP, DW), jnp.float32),
        jax.ShapeDtypeStruct((2, NP, DW), jnp.float32),
        jax.ShapeDtypeStruct((2, NP, 1), jnp.float32),
        jax.ShapeDtypeStruct((NP, 1), jnp.float32),
        jax.ShapeDtypeStruct((1, LANES), jnp.float32),
    ),
)


def _fin1_prep2_body(numa_ref, numb_ref, b1_ref, g1_ref, be1_ref,
                     w2_ref, as2_ref, ad2_ref,
                     h_out, inum_out, as_out, ad_out, maxs_out):
    num = jnp.concatenate([numa_ref[...][:N, :H // 2],
                           numb_ref[...][:N, :H // 2]], axis=1)
    den = numa_ref[...][:N, H // 2:H // 2 + 1]
    o = num / (den + 1e-16) + b1_ref[...].reshape(1, H)
    mu = jnp.mean(o, axis=0, keepdims=True)
    var = jnp.mean((o - mu) * (o - mu), axis=0, keepdims=True)
    o = (o - mu) / jnp.sqrt(var + 1e-5) * g1_ref[...].reshape(1, H) \
        + be1_ref[...].reshape(1, H)
    o = jnp.maximum(o, 0.0)
    h2 = jnp.dot(o, w2_ref[...], preferred_element_type=jnp.float32)
    a_s = jnp.dot(h2, as2_ref[...].reshape(H2, 1))
    a_d = jnp.dot(h2, ad2_ref[...].reshape(H2, 1))
    maxs = jnp.max(a_s)
    c = _lrelu(maxs + a_d)
    wself = jnp.exp(_lrelu(a_s + a_d) - c)
    h2f = _fold(h2)
    h_out[0] = h2f
    h_out[1] = h2f
    num0f = _fold_init(wself * h2, wself)
    inum_out[0] = num0f
    inum_out[1] = num0f
    asp = _pad_col(a_s)
    as_out[0] = asp
    as_out[1] = asp
    ad_out[...] = _pad_col(a_d)
    maxs_out[...] = jnp.full((1, LANES), maxs, jnp.float32)


_fin1_prep2 = pl.pallas_call(
    _fin1_prep2_body,
    compiler_params=pltpu.CompilerParams(vmem_limit_bytes=100 * 1024 * 1024),
    out_shape=(
        jax.ShapeDtypeStruct((2, NP, DW), jnp.float32),
        jax.ShapeDtypeStruct((2, NP, DW), jnp.float32),
        jax.ShapeDtypeStruct((2, NP, 1), jnp.float32),
        jax.ShapeDtypeStruct((NP, 1), jnp.float32),
        jax.ShapeDtypeStruct((1, LANES), jnp.float32),
    ),
)


def _fin2_body(numa_ref, b2_ref, g2_ref, be2_ref, batch_ref, wfc_ref,
               bfc_ref, out_ref):
    num = numa_ref[...][:N, :H2]
    den = numa_ref[...][:N, H2:H2 + 1]
    o = num / (den + 1e-16) + b2_ref[...].reshape(1, H2)
    mu = jnp.mean(o, axis=0, keepdims=True)
    var = jnp.mean((o - mu) * (o - mu), axis=0, keepdims=True)
    o = (o - mu) / jnp.sqrt(var + 1e-5) * g2_ref[...].reshape(1, H2) \
        + be2_ref[...].reshape(1, H2)
    o = jnp.maximum(o, 0.0)
    grp = lax.broadcasted_iota(jnp.int32, (N, G), 1)
    P = (batch_ref[...] == grp).astype(jnp.float32)
    cnum = ((0,), (0,)), ((), ())
    pooled = lax.dot_general(P, o, dimension_numbers=cnum,
                             preferred_element_type=jnp.float32)
    counts = lax.dot_general(P, jnp.ones((N, 1), jnp.float32),
                             dimension_numbers=cnum,
                             preferred_element_type=jnp.float32)
    pooled = pooled / jnp.maximum(counts, 1.0)
    out_ref[...] = jnp.dot(pooled, wfc_ref[...],
                           preferred_element_type=jnp.float32) \
        + bfc_ref[...].reshape(1, OUT)


_fin2 = pl.pallas_call(
    _fin2_body,
    compiler_params=pltpu.CompilerParams(vmem_limit_bytes=100 * 1024 * 1024),
    out_shape=jax.ShapeDtypeStruct((G, OUT), jnp.float32),
)


@functools.lru_cache(maxsize=None)
def _make_edge_kernel():
    rpt = NP // NS

    mesh = plsc.VectorSubcoreMesh(core_axis_name="c", subcore_axis_name="s",
                                  num_cores=NC, num_subcores=NS)

    @functools.partial(
        pl.kernel,
        out_type=jax.ShapeDtypeStruct((NC, NP, DW), jnp.float32),
        mesh=mesh,
        compiler_params=pltpu.CompilerParams(use_tc_tiling_on_sc=False),
        scratch_types=dict(
            sh_num=pltpu.VMEM_SHARED((NP, DW), jnp.float32),
            sh_as=pltpu.VMEM_SHARED((2 * NP,), jnp.float32),
            sh_ad=pltpu.VMEM_SHARED((NP,), jnp.float32),
            gidx=pltpu.VMEM((2, NSUB, SUB), jnp.int32),
            didx=pltpu.VMEM((2, NSUB, SUB), jnp.int32),
            asb=pltpu.VMEM((2, NSUB, SUB), jnp.float32),
            adb=pltpu.VMEM((2, NSUB, SUB), jnp.float32),
            wlin=pltpu.VMEM((2, CH), jnp.float32),
            maxs_t=pltpu.VMEM((LANES,), jnp.float32),
            rows=pltpu.VMEM((2, NSUB, SUB, DW), jnp.float32),
            sem_s0=pltpu.SemaphoreType.DMA,
            sem_s1=pltpu.SemaphoreType.DMA,
            sem_r0=pltpu.SemaphoreType.DMA,
            sem_r1=pltpu.SemaphoreType.DMA,
            sem_w0=pltpu.SemaphoreType.DMA,
            sem_w1=pltpu.SemaphoreType.DMA,
        ),
    )
    def edge_kernel(srcg, dst3d, h_hbm, as_hbm, ad_hbm, maxs_hbm,
                    inum_hbm, num_out,
                    sh_num, sh_as, sh_ad, gidx, didx, asb, adb,
                    wlin, maxs_t, rows, sem_s0, sem_s1, sem_r0, sem_r1,
                    sem_w0, sem_w1):
        cid = lax.axis_index("c")
        sid = lax.axis_index("s")
        r0 = sid * rpt

        pltpu.sync_copy(inum_hbm.at[cid, pl.ds(r0, rpt)],
                        sh_num.at[pl.ds(r0, rpt)])
        pltpu.sync_copy(as_hbm.at[pl.ds(r0, rpt)], sh_as.at[pl.ds(r0, rpt)])
        pltpu.sync_copy(as_hbm.at[pl.ds(NP + r0, rpt)],
                        sh_as.at[pl.ds(NP + r0, rpt)])
        pltpu.sync_copy(ad_hbm.at[pl.ds(r0, rpt)], sh_ad.at[pl.ds(r0, rpt)])
        pltpu.sync_copy(maxs_hbm, maxs_t)
        plsc.subcore_barrier()

        maxv = maxs_t[...]
        chunk0 = sid * NCHUNK

        def fetch(ci, p, sem_s, sem_r):
            pltpu.sync_copy(srcg.at[cid, ci], gidx.at[p])
            pltpu.sync_copy(dst3d.at[ci], didx.at[p])
            for j in range(NSUB):
                pltpu.async_copy(sh_as.at[gidx.at[p].at[j]],
                                 asb.at[p].at[j], sem_s)
                pltpu.async_copy(sh_ad.at[didx.at[p].at[j]],
                                 adb.at[p].at[j], sem_s)
                pltpu.async_copy(h_hbm.at[gidx.at[p].at[j]],
                                 rows.at[p].at[j], sem_r)

        def drain_scatter(p, sem_w):
            for j in range(NSUB):
                pltpu.make_async_copy(rows.at[p].at[j],
                                      sh_num.at[didx.at[p].at[j]],
                                      sem_w).wait()

        def compute(p, sem_s, sem_r, sem_w):
            for j in range(NSUB):
                pltpu.make_async_copy(sh_as.at[gidx.at[p].at[j]],
                                      asb.at[p].at[j], sem_s).wait()
                pltpu.make_async_copy(sh_ad.at[didx.at[p].at[j]],
                                      adb.at[p].at[j], sem_s).wait()
            for i in range(CH // LANES):
                r, off = i // (SUB // LANES), (i % (SUB // LANES)) * LANES
                asg = asb[p, r, pl.ds(off, LANES)]
                adg = adb[p, r, pl.ds(off, LANES)]
                e = asg + adg
                e = jnp.where(e > 0, e, 0.2 * e)
                cg = maxv + adg
                cg = jnp.where(cg > 0, cg, 0.2 * cg)
                w = jnp.exp(e - cg)
                wlin[p, pl.ds(i * LANES, LANES)] = w

            for j in range(NSUB):
                pltpu.make_async_copy(h_hbm.at[gidx.at[p].at[j]],
                                      rows.at[p].at[j], sem_r).wait()
            for j in range(NSUB):
                def scale_body(m, _):
                    wv = wlin[p, pl.ds(j * SUB + m * LANES, LANES)]
                    for l in range(LANES):
                        k = m * LANES + l
                        wsc = jnp.full((LANES,), wv[l], jnp.float32)
                        for q in range(DW // LANES):
                            sl = pl.ds(q * LANES, LANES)
                            rows[p, j, k, sl] = rows[p, j, k, sl] * wsc
                    return 0

                lax.fori_loop(0, SUB // LANES, scale_body, 0)

            for j in range(NSUB):
                pltpu.async_copy(rows.at[p].at[j],
                                 sh_num.at[didx.at[p].at[j]], sem_w,
                                 add=True)

        fetch(chunk0, 0, sem_s0, sem_r0)

        def pair_body(t, _):
            g = chunk0 + 2 * t
            fetch(g + 1, 1, sem_s1, sem_r1)
            compute(0, sem_s0, sem_r0, sem_w0)
            drain_scatter(0, sem_w0)

            @pl.when(t + 1 < NCHUNK // 2)
            def _():
                fetch(g + 2, 0, sem_s0, sem_r0)
            compute(1, sem_s1, sem_r1, sem_w1)
            drain_scatter(1, sem_w1)
            return 0

        lax.fori_loop(0, NCHUNK // 2, pair_body, 0)
        plsc.subcore_barrier()

        pltpu.sync_copy(sh_num.at[pl.ds(r0, rpt)],
                        num_out.at[cid, pl.ds(r0, rpt)])

    return edge_kernel


def kernel(x, edge_index, batch, W1, att_src1, att_dst1, b1, g1, be1,
           W2, att_src2, att_dst2, b2, g2, be2, Wfc, bfc):
    epad = jnp.full((E_PAD - E,), N, jnp.int32)
    src3d = jnp.concatenate([edge_index[0], epad]).reshape(E_PAD // CH,
                                                           NSUB, SUB)
    dst3d = jnp.concatenate([edge_index[1], epad]).reshape(E_PAD // CH,
                                                           NSUB, SUB)
    srcg = jnp.stack([src3d, src3d + NP])

    _edge = _make_edge_kernel()

    h1, inum1, as1, ad1, maxs1 = _prep1(x, W1, att_src1, att_dst1)
    num1 = _edge(srcg, dst3d, h1.reshape(2 * NP, DW),
                 as1.reshape(2 * NP), ad1.reshape(NP), maxs1.reshape(LANES),
                 inum1)
    h2, inum2, as2, ad2, maxs2 = _fin1_prep2(
        num1[0], num1[1], b1, g1, be1, W2, att_src2, att_dst2)
    num2 = _edge(srcg, dst3d, h2.reshape(2 * NP, DW),
                 as2.reshape(2 * NP), ad2.reshape(NP), maxs2.reshape(LANES),
                 inum2)
    out = _fin2(num2[0], b2, g2, be2, batch.reshape(N, 1), Wfc, bfc)
    return out
